# baseline (device time: 45051 ns/iter reference)
import jax
import jax.numpy as jnp
from jax import lax
from jax.experimental import pallas as pl
from jax.experimental.pallas import tpu as pltpu

N_DEV = 4
B, SQ, SKV = 2, 512, 512
HQ_LOC, DH = 8, 64
D_MODEL = 768
D_HID_LOC = HQ_LOC * DH
BLK = 64
ROWS = B * SQ
Q4 = ROWS // 4
Q8 = ROWS // 8


def kernel(x, Wq, K_ext, V_ext, Wo):
    K_t = jnp.transpose(K_ext, (0, 2, 1, 3)).astype(jnp.bfloat16)
    V_t = jnp.transpose(V_ext, (0, 2, 1, 3)).astype(jnp.bfloat16)
    x = x.astype(jnp.bfloat16)
    Wq = Wq.astype(jnp.bfloat16)
    Wo = Wo.astype(jnp.bfloat16)

    def body(x_ref, wq_ref, k_ref, v_ref, wo_ref, out_ref,
             ctx_ref, q_ref, bias_ref, sbuf1_ref, sbuf2_ref,
             rbuf1_ref, rbuf2_ref,
             sag1_ref, sag2_ref, rag1_ref, rag2_ref,
             send_sems, recv_sems):
        my = lax.axis_index("i")
        pu = jnp.bitwise_xor(my, 1)
        pv = 3 - my
        u = jnp.bitwise_and(jnp.bitwise_xor(my, my >> 1), 1)
        v = my >> 1

        barrier_sem = pltpu.get_barrier_semaphore()
        for nbr in (pu, pv):
            pl.semaphore_signal(
                barrier_sem, inc=1,
                device_id=(nbr,), device_id_type=pl.DeviceIdType.MESH,
            )
        pl.semaphore_wait(barrier_sem, 2)

        k1 = u * Q4
        s1 = (1 - u) * Q4
        k2 = 2 * Q4 + v * Q4
        s2 = 2 * Q4 + (1 - v) * Q4
        h1k = k1 + v * Q8
        h1s = k1 + (1 - v) * Q8
        h2k = k2 + u * Q8
        h2s = k2 + (1 - u) * Q8

        xm = x_ref[:].reshape(ROWS, D_MODEL)
        wq = wq_ref[:, pl.ds(my * D_HID_LOC, D_HID_LOC)]
        q_ref[:] = jnp.dot(
            xm, wq, preferred_element_type=jnp.float32
        ).astype(jnp.bfloat16)
        wo = wo_ref[pl.ds(my * D_HID_LOC, D_HID_LOC), :]

        qb = lax.broadcasted_iota(jnp.int32, (SQ, SKV), 0) // BLK
        kb = lax.broadcasted_iota(jnp.int32, (SQ, SKV), 1) // BLK
        mask = (qb == kb) | (kb == 0) | (lax.rem(qb + kb, 3) == 0)
        bias_ref[:] = jnp.where(mask, 0.0, -1e9).astype(jnp.float32)

        def quarter(b, o):
            row0 = b * SQ + o
            mo = bias_ref[pl.ds(o, Q4), :]
            for h in range(HQ_LOC):
                qh = q_ref[pl.ds(row0, Q4), h * DH:(h + 1) * DH]
                kh = k_ref[b, h, :, :]
                vh = v_ref[b, h, :, :]
                s = lax.dot_general(
                    qh, kh, (((1,), (1,)), ((), ())),
                    preferred_element_type=jnp.float32,
                ) * 0.125 + mo
                m = jnp.max(s, axis=1, keepdims=True)
                w = jnp.exp(s - m)
                w = (w / jnp.sum(w, axis=1, keepdims=True)).astype(
                    jnp.bfloat16)
                ctx_ref[pl.ds(row0, Q4), h * DH:(h + 1) * DH] = jnp.dot(
                    w, vh, preferred_element_type=jnp.float32
                ).astype(jnp.bfloat16)
            out_ref[pl.ds(row0, Q4), :] = jnp.dot(
                ctx_ref[pl.ds(row0, Q4), :], wo,
                preferred_element_type=jnp.float32)

        def xchg(e, src, dst, peer):
            r = pltpu.make_async_remote_copy(
                src_ref=src, dst_ref=dst,
                send_sem=send_sems.at[e], recv_sem=recv_sems.at[e],
                device_id=(peer,), device_id_type=pl.DeviceIdType.MESH,
            )
            r.start()
            return r

        quarter(0, s1)
        sbuf1_ref[0] = out_ref[pl.ds(s1, Q4), :].astype(jnp.bfloat16)
        r1 = xchg(0, sbuf1_ref.at[0], rbuf1_ref.at[0], pu)
        quarter(1, s2 - SQ)
        sbuf1_ref[1] = out_ref[pl.ds(s2, Q4), :].astype(jnp.bfloat16)
        r2 = xchg(1, sbuf1_ref.at[1], rbuf1_ref.at[1], pv)
        quarter(0, k1)
        quarter(1, k2 - SQ)
        r1.wait()
        r2.wait()
        out_ref[pl.ds(k1, Q4), :] = (
            out_ref[pl.ds(k1, Q4), :] + rbuf1_ref[0].astype(jnp.float32))
        out_ref[pl.ds(k2, Q4), :] = (
            out_ref[pl.ds(k2, Q4), :] + rbuf1_ref[1].astype(jnp.float32))

        sbuf2_ref[0] = out_ref[pl.ds(h1s, Q8), :].astype(jnp.bfloat16)
        sbuf2_ref[1] = out_ref[pl.ds(h2s, Q8), :].astype(jnp.bfloat16)
        r1 = xchg(2, sbuf2_ref.at[0], rbuf2_ref.at[0], pv)
        r2 = xchg(3, sbuf2_ref.at[1], rbuf2_ref.at[1], pu)
        r1.wait()
        r2.wait()
        out_ref[pl.ds(h1k, Q8), :] = (
            out_ref[pl.ds(h1k, Q8), :] + rbuf2_ref[0].astype(jnp.float32))
        out_ref[pl.ds(h2k, Q8), :] = (
            out_ref[pl.ds(h2k, Q8), :] + rbuf2_ref[1].astype(jnp.float32))

        sag2_ref[0] = out_ref[pl.ds(h1k, Q8), :].astype(jnp.bfloat16)
        sag2_ref[1] = out_ref[pl.ds(h2k, Q8), :].astype(jnp.bfloat16)
        r1 = xchg(4, sag2_ref.at[0], rag2_ref.at[0], pv)
        r2 = xchg(5, sag2_ref.at[1], rag2_ref.at[1], pu)
        r1.wait()
        r2.wait()
        out_ref[pl.ds(h1s, Q8), :] = rag2_ref[0].astype(jnp.float32)
        out_ref[pl.ds(h2s, Q8), :] = rag2_ref[1].astype(jnp.float32)

        sag1_ref[0] = out_ref[pl.ds(k1, Q4), :].astype(jnp.bfloat16)
        sag1_ref[1] = out_ref[pl.ds(k2, Q4), :].astype(jnp.bfloat16)
        r1 = xchg(6, sag1_ref.at[0], rag1_ref.at[0], pu)
        r2 = xchg(7, sag1_ref.at[1], rag1_ref.at[1], pv)
        r1.wait()
        r2.wait()
        out_ref[pl.ds(s1, Q4), :] = rag1_ref[0].astype(jnp.float32)
        out_ref[pl.ds(s2, Q4), :] = rag1_ref[1].astype(jnp.float32)

    out = pl.pallas_call(
        body,
        out_shape=jax.ShapeDtypeStruct((ROWS, D_MODEL), jnp.float32),
        in_specs=[pl.BlockSpec(memory_space=pltpu.VMEM)] * 5,
        out_specs=pl.BlockSpec(memory_space=pltpu.VMEM),
        scratch_shapes=[
            pltpu.VMEM((ROWS, D_HID_LOC), jnp.bfloat16),
            pltpu.VMEM((ROWS, D_HID_LOC), jnp.bfloat16),
            pltpu.VMEM((SQ, SKV), jnp.float32),
            pltpu.VMEM((2, Q4, D_MODEL), jnp.bfloat16),
            pltpu.VMEM((2, Q8, D_MODEL), jnp.bfloat16),
            pltpu.VMEM((2, Q4, D_MODEL), jnp.bfloat16),
            pltpu.VMEM((2, Q8, D_MODEL), jnp.bfloat16),
            pltpu.VMEM((2, Q4, D_MODEL), jnp.bfloat16),
            pltpu.VMEM((2, Q8, D_MODEL), jnp.bfloat16),
            pltpu.VMEM((2, Q4, D_MODEL), jnp.bfloat16),
            pltpu.VMEM((2, Q8, D_MODEL), jnp.bfloat16),
            pltpu.SemaphoreType.DMA((8,)),
            pltpu.SemaphoreType.DMA((8,)),
        ],
        compiler_params=pltpu.CompilerParams(collective_id=0),
    )(x, Wq, K_t, V_t, Wo)
    return out.reshape(B, SQ, D_MODEL)


# device time: 43772 ns/iter; 1.0292x vs baseline; 1.0292x over previous
import jax
import jax.numpy as jnp
from jax import lax
from jax.experimental import pallas as pl
from jax.experimental.pallas import tpu as pltpu

N_DEV = 4
B, SQ, SKV = 2, 512, 512
HQ_LOC, DH = 8, 64
D_MODEL = 768
D_HID_LOC = HQ_LOC * DH
BLK = 64
ROWS = B * SQ
Q4 = ROWS // 4
Q8 = ROWS // 8


def kernel(x, Wq, K_ext, V_ext, Wo):
    K_t = jnp.transpose(K_ext, (0, 2, 1, 3))
    V_t = jnp.transpose(V_ext, (0, 2, 1, 3))

    def body(x_ref, wq_ref, k_ref, v_ref, wo_ref, out_ref,
             ctx_ref, q_ref, bias_ref, sbuf1_ref, sbuf2_ref,
             rbuf1_ref, rbuf2_ref,
             sag1_ref, sag2_ref, rag1_ref, rag2_ref,
             send_sems, recv_sems):
        my = lax.axis_index("i")
        pu = jnp.bitwise_xor(my, 1)
        pv = 3 - my
        u = jnp.bitwise_and(jnp.bitwise_xor(my, my >> 1), 1)
        v = my >> 1

        barrier_sem = pltpu.get_barrier_semaphore()
        for nbr in (pu, pv):
            pl.semaphore_signal(
                barrier_sem, inc=1,
                device_id=(nbr,), device_id_type=pl.DeviceIdType.MESH,
            )
        pl.semaphore_wait(barrier_sem, 2)

        k1 = u * Q4
        s1 = (1 - u) * Q4
        k2 = 2 * Q4 + v * Q4
        s2 = 2 * Q4 + (1 - v) * Q4
        h1k = k1 + v * Q8
        h1s = k1 + (1 - v) * Q8
        h2k = k2 + u * Q8
        h2s = k2 + (1 - u) * Q8

        xm = x_ref[:].reshape(ROWS, D_MODEL).astype(jnp.bfloat16)
        wq = wq_ref[:, pl.ds(my * D_HID_LOC, D_HID_LOC)].astype(jnp.bfloat16)
        q_ref[:] = jnp.dot(
            xm, wq, preferred_element_type=jnp.float32
        ).astype(jnp.bfloat16)
        wo = wo_ref[pl.ds(my * D_HID_LOC, D_HID_LOC), :].astype(jnp.bfloat16)

        qb = lax.broadcasted_iota(jnp.int32, (SQ, SKV), 0) // BLK
        kb = lax.broadcasted_iota(jnp.int32, (SQ, SKV), 1) // BLK
        mask = (qb == kb) | (kb == 0) | (lax.rem(qb + kb, 3) == 0)
        bias_ref[:] = jnp.where(mask, 0.0, -1e9).astype(jnp.float32)

        def quarter(b, o):
            row0 = b * SQ + o
            mo = bias_ref[pl.ds(o, Q4), :]
            for h in range(HQ_LOC):
                qh = q_ref[pl.ds(row0, Q4), h * DH:(h + 1) * DH]
                kh = k_ref[b, h, :, :].astype(jnp.bfloat16)
                vh = v_ref[b, h, :, :].astype(jnp.bfloat16)
                s = lax.dot_general(
                    qh, kh, (((1,), (1,)), ((), ())),
                    preferred_element_type=jnp.float32,
                ) * 0.125 + mo
                m = jnp.max(s, axis=1, keepdims=True)
                w = jnp.exp(s - m)
                w = (w / jnp.sum(w, axis=1, keepdims=True)).astype(
                    jnp.bfloat16)
                ctx_ref[pl.ds(row0, Q4), h * DH:(h + 1) * DH] = jnp.dot(
                    w, vh, preferred_element_type=jnp.float32
                ).astype(jnp.bfloat16)
            out_ref[pl.ds(row0, Q4), :] = jnp.dot(
                ctx_ref[pl.ds(row0, Q4), :], wo,
                preferred_element_type=jnp.float32)

        def xchg(e, src, dst, peer):
            r = pltpu.make_async_remote_copy(
                src_ref=src, dst_ref=dst,
                send_sem=send_sems.at[e], recv_sem=recv_sems.at[e],
                device_id=(peer,), device_id_type=pl.DeviceIdType.MESH,
            )
            r.start()
            return r

        quarter(0, s1)
        sbuf1_ref[0] = out_ref[pl.ds(s1, Q4), :].astype(jnp.bfloat16)
        r1 = xchg(0, sbuf1_ref.at[0], rbuf1_ref.at[0], pu)
        quarter(1, s2 - SQ)
        sbuf1_ref[1] = out_ref[pl.ds(s2, Q4), :].astype(jnp.bfloat16)
        r2 = xchg(1, sbuf1_ref.at[1], rbuf1_ref.at[1], pv)
        quarter(0, k1)
        quarter(1, k2 - SQ)
        r1.wait()
        r2.wait()
        out_ref[pl.ds(k1, Q4), :] = (
            out_ref[pl.ds(k1, Q4), :] + rbuf1_ref[0].astype(jnp.float32))
        out_ref[pl.ds(k2, Q4), :] = (
            out_ref[pl.ds(k2, Q4), :] + rbuf1_ref[1].astype(jnp.float32))

        sbuf2_ref[0] = out_ref[pl.ds(h1s, Q8), :].astype(jnp.bfloat16)
        sbuf2_ref[1] = out_ref[pl.ds(h2s, Q8), :].astype(jnp.bfloat16)
        r1 = xchg(2, sbuf2_ref.at[0], rbuf2_ref.at[0], pv)
        r2 = xchg(3, sbuf2_ref.at[1], rbuf2_ref.at[1], pu)
        r1.wait()
        r2.wait()
        out_ref[pl.ds(h1k, Q8), :] = (
            out_ref[pl.ds(h1k, Q8), :] + rbuf2_ref[0].astype(jnp.float32))
        out_ref[pl.ds(h2k, Q8), :] = (
            out_ref[pl.ds(h2k, Q8), :] + rbuf2_ref[1].astype(jnp.float32))

        sag2_ref[0] = out_ref[pl.ds(h1k, Q8), :].astype(jnp.bfloat16)
        sag2_ref[1] = out_ref[pl.ds(h2k, Q8), :].astype(jnp.bfloat16)
        r1 = xchg(4, sag2_ref.at[0], rag2_ref.at[0], pv)
        r2 = xchg(5, sag2_ref.at[1], rag2_ref.at[1], pu)
        r1.wait()
        r2.wait()
        out_ref[pl.ds(h1s, Q8), :] = rag2_ref[0].astype(jnp.float32)
        out_ref[pl.ds(h2s, Q8), :] = rag2_ref[1].astype(jnp.float32)

        sag1_ref[0] = out_ref[pl.ds(k1, Q4), :].astype(jnp.bfloat16)
        sag1_ref[1] = out_ref[pl.ds(k2, Q4), :].astype(jnp.bfloat16)
        r1 = xchg(6, sag1_ref.at[0], rag1_ref.at[0], pu)
        r2 = xchg(7, sag1_ref.at[1], rag1_ref.at[1], pv)
        r1.wait()
        r2.wait()
        out_ref[pl.ds(s1, Q4), :] = rag1_ref[0].astype(jnp.float32)
        out_ref[pl.ds(s2, Q4), :] = rag1_ref[1].astype(jnp.float32)

    out = pl.pallas_call(
        body,
        out_shape=jax.ShapeDtypeStruct((ROWS, D_MODEL), jnp.float32),
        in_specs=[pl.BlockSpec(memory_space=pltpu.VMEM)] * 5,
        out_specs=pl.BlockSpec(memory_space=pltpu.VMEM),
        scratch_shapes=[
            pltpu.VMEM((ROWS, D_HID_LOC), jnp.bfloat16),
            pltpu.VMEM((ROWS, D_HID_LOC), jnp.bfloat16),
            pltpu.VMEM((SQ, SKV), jnp.float32),
            pltpu.VMEM((2, Q4, D_MODEL), jnp.bfloat16),
            pltpu.VMEM((2, Q8, D_MODEL), jnp.bfloat16),
            pltpu.VMEM((2, Q4, D_MODEL), jnp.bfloat16),
            pltpu.VMEM((2, Q8, D_MODEL), jnp.bfloat16),
            pltpu.VMEM((2, Q4, D_MODEL), jnp.bfloat16),
            pltpu.VMEM((2, Q8, D_MODEL), jnp.bfloat16),
            pltpu.VMEM((2, Q4, D_MODEL), jnp.bfloat16),
            pltpu.VMEM((2, Q8, D_MODEL), jnp.bfloat16),
            pltpu.SemaphoreType.DMA((8,)),
            pltpu.SemaphoreType.DMA((8,)),
        ],
        compiler_params=pltpu.CompilerParams(collective_id=0),
    )(x, Wq, K_t, V_t, Wo)
    return out.reshape(B, SQ, D_MODEL)


# device time: 35669 ns/iter; 1.2630x vs baseline; 1.2272x over previous
import jax
import jax.numpy as jnp
from jax import lax
from jax.experimental import pallas as pl
from jax.experimental.pallas import tpu as pltpu

N_DEV = 4
B, SQ, SKV = 2, 512, 512
HQ_LOC, DH = 8, 64
D_MODEL = 768
D_HID_LOC = HQ_LOC * DH
BLK = 64
ROWS = B * SQ
_COMM = True
Q4 = ROWS // 4
Q8 = ROWS // 8


def kernel(x, Wq, K_ext, V_ext, Wo):
    K_t = jnp.transpose(K_ext, (0, 2, 1, 3))
    V_t = jnp.transpose(V_ext, (0, 2, 1, 3))

    def body(x_ref, wq_ref, k_ref, v_ref, wo_ref, out_ref,
             ctx_ref, q_ref, bias_ref, wq_v, wo_v,
             rbuf1_ref, rbuf2_ref,
             send_sems, recv_sems, wsem):
        my = lax.axis_index("i")
        pu = jnp.bitwise_xor(my, 1)
        pv = 3 - my
        dg = 3 - jnp.bitwise_xor(my, 1)
        u = jnp.bitwise_and(jnp.bitwise_xor(my, my >> 1), 1)
        v = my >> 1

        cp_wq = pltpu.make_async_copy(
            wq_ref.at[:, pl.ds(my * D_HID_LOC, D_HID_LOC)], wq_v,
            wsem.at[0])
        cp_wq.start()
        cp_wo = pltpu.make_async_copy(
            wo_ref.at[pl.ds(my * D_HID_LOC, D_HID_LOC), :], wo_v,
            wsem.at[1])
        cp_wo.start()

        barrier_sem = pltpu.get_barrier_semaphore()
        for nbr in (pu, pv, dg):
            pl.semaphore_signal(
                barrier_sem, inc=1,
                device_id=(nbr,), device_id_type=pl.DeviceIdType.MESH,
            )
        pl.semaphore_wait(barrier_sem, 3)

        k1 = u * Q4
        s1 = (1 - u) * Q4
        k2 = 2 * Q4 + v * Q4
        s2 = 2 * Q4 + (1 - v) * Q4
        h1k = k1 + v * Q8
        h1s = k1 + (1 - v) * Q8
        h2k = k2 + u * Q8
        h2s = k2 + (1 - u) * Q8

        qb = lax.broadcasted_iota(jnp.int32, (SQ, SKV), 0) // BLK
        kb = lax.broadcasted_iota(jnp.int32, (SQ, SKV), 1) // BLK
        mask = (qb == kb) | (kb == 0) | (lax.rem(qb + kb, 3) == 0)
        bias_ref[:] = jnp.where(mask, 0.0, -1e9).astype(jnp.float32)

        cp_wq.wait()
        xm = x_ref[:].reshape(ROWS, D_MODEL)
        q_ref[:] = jnp.dot(
            xm, wq_v[:], preferred_element_type=jnp.float32) * 0.125
        cp_wo.wait()

        def block(b, o, n):
            row0 = b * SQ + o
            mo = bias_ref[pl.ds(o, n), :]
            for h in range(HQ_LOC):
                qh = q_ref[pl.ds(row0, n), h * DH:(h + 1) * DH]
                kh = k_ref[b, h, :, :]
                vh = v_ref[b, h, :, :]
                s = lax.dot_general(
                    qh, kh, (((1,), (1,)), ((), ())),
                    preferred_element_type=jnp.float32,
                ) + mo
                w = jnp.exp(s)
                denom = jnp.sum(w, axis=1, keepdims=True)
                ctx_ref[pl.ds(row0, n), h * DH:(h + 1) * DH] = jnp.dot(
                    w, vh, preferred_element_type=jnp.float32) / denom
            out_ref[pl.ds(row0, n), :] = jnp.dot(
                ctx_ref[pl.ds(row0, n), :], wo_v[:],
                preferred_element_type=jnp.float32).astype(jnp.bfloat16)

        def xchg(e, src, dst, peer):
            r = pltpu.make_async_remote_copy(
                src_ref=src, dst_ref=dst,
                send_sem=send_sems.at[e], recv_sem=recv_sems.at[e],
                device_id=(peer,), device_id_type=pl.DeviceIdType.MESH,
            )
            r.start()
            return r

        block(0, s1, Q4)
        if _COMM:
            rp1b1 = xchg(0, out_ref.at[pl.ds(s1, Q4), :],
                         rbuf1_ref.at[0], pu)
        block(1, s2 - SQ, Q4)
        if _COMM:
            rp1b2 = xchg(1, out_ref.at[pl.ds(s2, Q4), :],
                         rbuf1_ref.at[1], pv)
        block(0, k1, Q4)
        if _COMM:
            rp1b1.wait()
            out_ref[pl.ds(h1s, Q8), :] = (
                out_ref[pl.ds(h1s, Q8), :]
                + rbuf1_ref[0, pl.ds((1 - v) * Q8, Q8), :])
            rp2b1 = xchg(2, out_ref.at[pl.ds(h1s, Q8), :],
                         rbuf2_ref.at[0], pv)
            out_ref[pl.ds(h1k, Q8), :] = (
                out_ref[pl.ds(h1k, Q8), :]
                + rbuf1_ref[0, pl.ds(v * Q8, Q8), :])
        block(1, k2 - SQ, Q4)
        if not _COMM:
            return
        rp1b2.wait()
        out_ref[pl.ds(h2s, Q8), :] = (
            out_ref[pl.ds(h2s, Q8), :]
            + rbuf1_ref[1, pl.ds((1 - u) * Q8, Q8), :])
        rp2b2 = xchg(3, out_ref.at[pl.ds(h2s, Q8), :],
                     rbuf2_ref.at[1], pu)
        out_ref[pl.ds(h2k, Q8), :] = (
            out_ref[pl.ds(h2k, Q8), :]
            + rbuf1_ref[1, pl.ds(u * Q8, Q8), :])

        rp2b1.wait()
        out_ref[pl.ds(h1k, Q8), :] = (
            out_ref[pl.ds(h1k, Q8), :] + rbuf2_ref[0])
        b1 = out_ref.at[pl.ds(h1k, Q8), :]
        ag = [xchg(4, b1, b1, pu), xchg(5, b1, b1, pv), xchg(6, b1, b1, dg)]
        rp2b2.wait()
        out_ref[pl.ds(h2k, Q8), :] = (
            out_ref[pl.ds(h2k, Q8), :] + rbuf2_ref[1])
        b2 = out_ref.at[pl.ds(h2k, Q8), :]
        ag += [xchg(7, b2, b2, pu), xchg(8, b2, b2, pv), xchg(9, b2, b2, dg)]
        for r in ag:
            r.wait()

    out = pl.pallas_call(
        body,
        out_shape=jax.ShapeDtypeStruct((ROWS, D_MODEL), jnp.bfloat16),
        in_specs=[
            pl.BlockSpec(memory_space=pltpu.VMEM),
            pl.BlockSpec(memory_space=pltpu.MemorySpace.HBM),
            pl.BlockSpec(memory_space=pltpu.VMEM),
            pl.BlockSpec(memory_space=pltpu.VMEM),
            pl.BlockSpec(memory_space=pltpu.MemorySpace.HBM),
        ],
        out_specs=pl.BlockSpec(memory_space=pltpu.VMEM),
        scratch_shapes=[
            pltpu.VMEM((ROWS, D_HID_LOC), jnp.float32),
            pltpu.VMEM((ROWS, D_HID_LOC), jnp.float32),
            pltpu.VMEM((SQ, SKV), jnp.float32),
            pltpu.VMEM((D_MODEL, D_HID_LOC), jnp.float32),
            pltpu.VMEM((D_HID_LOC, D_MODEL), jnp.float32),
            pltpu.VMEM((2, Q4, D_MODEL), jnp.bfloat16),
            pltpu.VMEM((2, Q8, D_MODEL), jnp.bfloat16),
            pltpu.SemaphoreType.DMA((10,)),
            pltpu.SemaphoreType.DMA((10,)),
            pltpu.SemaphoreType.DMA((2,)),
        ],
        compiler_params=pltpu.CompilerParams(collective_id=0),
    )(x, Wq, K_t, V_t, Wo)
    return out.reshape(B, SQ, D_MODEL)
